# baseline (speedup 1.0000x reference)
import functools

import jax
import jax.numpy as jnp
from jax import lax
from jax.experimental import pallas as pl
from jax.experimental.pallas import tpu as pltpu
from jax.experimental.pallas import tpu_sc as plsc

NUM_LAYERS = 5
D = 128
N_NODES = 10000
N_EDGES = 320000
NUM_GRAPHS = 64
BN_EPS = 1e-5

NC = 2
NS = 16
NW = NC * NS
CHUNK = 128
KSLOT = 2
CPT = 80
HALF = CPT // 2
EDGES_PAD = NW * CPT * CHUNK
RPT = 632
N_ACC = NS * RPT
RPT_LAST = N_NODES - (NS - 1) * RPT


def _agg_body(h_hbm, row_hbm, col_hbm, zeros_hbm, out_hbm,
              row_v, col_v, rows_v, acc_sh, *sems):
    cid = lax.axis_index("c")
    sid = lax.axis_index("s")
    wid = cid * NS + sid
    pltpu.sync_copy(zeros_hbm.at[pl.ds(sid * RPT, RPT)],
                    acc_sh.at[pl.ds(sid * RPT, RPT)])

    for ph in range(CPT // HALF):
        pltpu.sync_copy(row_hbm.at[wid, pl.ds(ph * HALF, HALF)], row_v)
        pltpu.sync_copy(col_hbm.at[wid, pl.ds(ph * HALF, HALF)], col_v)
        for s in range(2):
            pltpu.async_copy(h_hbm.at[col_v.at[s]], rows_v.at[s], sems[s])
        if ph == 0:
            plsc.subcore_barrier()

        def blk(b, carry):
            for s in range(2):
                j = b * 2 + s
                pltpu.make_async_copy(h_hbm.at[col_v.at[j]], rows_v.at[s],
                                      sems[s]).wait()
                pltpu.sync_copy(rows_v.at[s], acc_sh.at[row_v.at[j]],
                                add=True)
                nxt = j + 2

                @pl.when(nxt < HALF)
                def _():
                    pltpu.async_copy(h_hbm.at[col_v.at[nxt]], rows_v.at[s],
                                     sems[s])
            return carry

        lax.fori_loop(0, HALF // 2, blk, 0)

    plsc.subcore_barrier()
    @pl.when(sid < NS - 1)
    def _():
        pltpu.sync_copy(acc_sh.at[pl.ds(sid * RPT, RPT)],
                        out_hbm.at[cid, pl.ds(sid * RPT, RPT)])

    @pl.when(sid == NS - 1)
    def _():
        pltpu.sync_copy(acc_sh.at[pl.ds((NS - 1) * RPT, RPT_LAST)],
                        out_hbm.at[cid, pl.ds((NS - 1) * RPT, RPT_LAST)])


@functools.cache
def _get_agg():
    return pl.kernel(
        _agg_body,
        out_type=jax.ShapeDtypeStruct((NC, N_NODES, D), jnp.float32),
        mesh=plsc.VectorSubcoreMesh(core_axis_name="c", subcore_axis_name="s"),
        scratch_types=[
            pltpu.VMEM((HALF, CHUNK), jnp.int32),
            pltpu.VMEM((HALF, CHUNK), jnp.int32),
            pltpu.VMEM((2, CHUNK, D), jnp.float32),
            pltpu.VMEM_SHARED((N_ACC, D), jnp.float32),
            pltpu.SemaphoreType.DMA,
            pltpu.SemaphoreType.DMA,
        ],
    )


def _bn_relu(t, g, be):
    mean = jnp.mean(t, axis=0, keepdims=True)
    c = t - mean
    var = jnp.mean(c * c, axis=0, keepdims=True)
    return jnp.maximum(g * (c * lax.rsqrt(var + BN_EPS)) + be, 0.0)


def _mlp_body(eps_ref, acc_ref, h_ref, w1_ref, b1_ref, w2_ref, b2_ref,
              g1_ref, be1_ref, g2_ref, be2_ref, batch_ref, hout_ref,
              pool_ref):
    scale = 1.0 + eps_ref[0, 0]
    pooled = acc_ref[0] + acc_ref[1] + scale * h_ref[...]
    t = lax.dot_general(pooled, w1_ref[...], (((1,), (0,)), ((), ())),
                        precision=lax.Precision.HIGHEST,
                        preferred_element_type=jnp.float32) + b1_ref[...]
    t = _bn_relu(t, g1_ref[...], be1_ref[...])
    t = lax.dot_general(t, w2_ref[...], (((1,), (0,)), ((), ())),
                        precision=lax.Precision.HIGHEST,
                        preferred_element_type=jnp.float32) + b2_ref[...]
    h_new = _bn_relu(t, g2_ref[...], be2_ref[...])
    hout_ref[...] = h_new
    onehot = (batch_ref[...] == lax.broadcasted_iota(
        jnp.int32, (N_NODES, NUM_GRAPHS), 1)).astype(jnp.float32)
    pool_ref[...] = lax.dot_general(onehot, h_new,
                                    (((0,), (0,)), ((), ())),
                                    precision=lax.Precision.HIGHEST,
                                    preferred_element_type=jnp.float32)


_mlp = pl.pallas_call(
    _mlp_body,
    out_shape=[
        jax.ShapeDtypeStruct((N_NODES, D), jnp.float32),
        jax.ShapeDtypeStruct((NUM_GRAPHS, D), jnp.float32),
    ],
    in_specs=[pl.BlockSpec(memory_space=pltpu.SMEM)]
    + [pl.BlockSpec(memory_space=pltpu.VMEM)] * 11,
)


def kernel(x, edge_index, batch, W1, b1, W2, b2, g1, be1, g2, be2, eps_param):
    row = edge_index[0].astype(jnp.int32)
    col = edge_index[1].astype(jnp.int32)
    pad = EDGES_PAD - N_EDGES
    pad_rows = N_NODES + jnp.arange(pad, dtype=jnp.int32) % (N_ACC - N_NODES)
    pad_cols = jnp.arange(pad, dtype=jnp.int32) % N_NODES
    row_r = jnp.concatenate([row, pad_rows]).reshape(CPT, NW, CHUNK
                                                    ).transpose(1, 0, 2)
    col_r = jnp.concatenate([col, pad_cols]).reshape(CPT, NW, CHUNK
                                                     ).transpose(1, 0, 2)
    zeros = jnp.zeros((N_ACC, D), jnp.float32)
    batch2d = batch.astype(jnp.int32).reshape(N_NODES, 1)

    h = x
    pools = []
    for l in range(NUM_LAYERS):
        acc = _get_agg()(h, row_r, col_r, zeros)
        h, pg = _mlp(eps_param[l].reshape(1, 1), acc, h,
                     W1[l], b1[l].reshape(1, D), W2[l], b2[l].reshape(1, D),
                     g1[l].reshape(1, D), be1[l].reshape(1, D),
                     g2[l].reshape(1, D), be2[l].reshape(1, D), batch2d)
        pools.append(pg)
    return jnp.concatenate(pools, axis=-1)


# speedup vs baseline: 1.0127x; 1.0127x over previous
import functools

import jax
import jax.numpy as jnp
from jax import lax
from jax.experimental import pallas as pl
from jax.experimental.pallas import tpu as pltpu
from jax.experimental.pallas import tpu_sc as plsc

NUM_LAYERS = 5
D = 128
N_NODES = 10000
N_EDGES = 320000
NUM_GRAPHS = 64
BN_EPS = 1e-5

NC = 2
NS = 16
NW = NC * NS
CHUNK = 128
KSLOT = 2
CPT = 80
HALF = CPT // 2
EDGES_PAD = NW * CPT * CHUNK
RPT = 632
N_ACC = NS * RPT
RPT_LAST = N_NODES - (NS - 1) * RPT


def _agg_body(h_hbm, row_hbm, col_hbm, zeros_hbm, out_hbm,
              row_v, col_v, rows_v, acc_sh, *sems):
    cid = lax.axis_index("c")
    sid = lax.axis_index("s")
    wid = cid * NS + sid
    pltpu.sync_copy(zeros_hbm.at[pl.ds(sid * RPT, RPT)],
                    acc_sh.at[pl.ds(sid * RPT, RPT)])

    for ph in range(CPT // HALF):
        pltpu.sync_copy(row_hbm.at[wid, pl.ds(ph * HALF, HALF)], row_v)
        pltpu.sync_copy(col_hbm.at[wid, pl.ds(ph * HALF, HALF)], col_v)
        for s in range(2):
            pltpu.async_copy(h_hbm.at[col_v.at[s]], rows_v.at[s], sems[s])
        if ph == 0:
            plsc.subcore_barrier()

        def blk(b, carry):
            for s in range(2):
                j = b * 2 + s
                pltpu.make_async_copy(h_hbm.at[col_v.at[j]], rows_v.at[s],
                                      sems[s]).wait()
                pltpu.sync_copy(rows_v.at[s], acc_sh.at[row_v.at[j]],
                                add=True)
                nxt = j + 2

                @pl.when(nxt < HALF)
                def _():
                    pltpu.async_copy(h_hbm.at[col_v.at[nxt]], rows_v.at[s],
                                     sems[s])
            return carry

        lax.fori_loop(0, HALF // 2, blk, 0)

    plsc.subcore_barrier()
    @pl.when(sid < NS - 1)
    def _():
        pltpu.sync_copy(acc_sh.at[pl.ds(sid * RPT, RPT)],
                        out_hbm.at[cid, pl.ds(sid * RPT, RPT)])

    @pl.when(sid == NS - 1)
    def _():
        pltpu.sync_copy(acc_sh.at[pl.ds((NS - 1) * RPT, RPT_LAST)],
                        out_hbm.at[cid, pl.ds((NS - 1) * RPT, RPT_LAST)])


@functools.cache
def _get_agg():
    return pl.kernel(
        _agg_body,
        out_type=jax.ShapeDtypeStruct((NC, N_NODES, D), jnp.float32),
        mesh=plsc.VectorSubcoreMesh(core_axis_name="c", subcore_axis_name="s"),
        scratch_types=[
            pltpu.VMEM((HALF, CHUNK), jnp.int32),
            pltpu.VMEM((HALF, CHUNK), jnp.int32),
            pltpu.VMEM((2, CHUNK, D), jnp.float32),
            pltpu.VMEM_SHARED((N_ACC, D), jnp.float32),
            pltpu.SemaphoreType.DMA,
            pltpu.SemaphoreType.DMA,
        ],
    )


def _bn_relu(t, g, be):
    mean = jnp.mean(t, axis=0, keepdims=True)
    c = t - mean
    var = jnp.mean(c * c, axis=0, keepdims=True)
    return jnp.maximum(g * (c * lax.rsqrt(var + BN_EPS)) + be, 0.0)


def _mlp_body(eps_ref, acc_ref, h_ref, w1_ref, b1_ref, w2_ref, b2_ref,
              g1_ref, be1_ref, g2_ref, be2_ref, hout_ref):
    scale = 1.0 + eps_ref[0, 0]
    pooled = acc_ref[0] + acc_ref[1] + scale * h_ref[...]
    t = lax.dot_general(pooled, w1_ref[...], (((1,), (0,)), ((), ())),
                        precision=lax.Precision.HIGHEST,
                        preferred_element_type=jnp.float32) + b1_ref[...]
    t = _bn_relu(t, g1_ref[...], be1_ref[...])
    t = lax.dot_general(t, w2_ref[...], (((1,), (0,)), ((), ())),
                        precision=lax.Precision.HIGHEST,
                        preferred_element_type=jnp.float32) + b2_ref[...]
    hout_ref[...] = _bn_relu(t, g2_ref[...], be2_ref[...])


_mlp = pl.pallas_call(
    _mlp_body,
    out_shape=jax.ShapeDtypeStruct((N_NODES, D), jnp.float32),
    in_specs=[pl.BlockSpec(memory_space=pltpu.SMEM)]
    + [pl.BlockSpec(memory_space=pltpu.VMEM)] * 10,
)


def _pool_body(batch_ref, h_ref, pool_ref):
    onehot = (batch_ref[...] == lax.broadcasted_iota(
        jnp.int32, (N_NODES, NUM_GRAPHS), 1)).astype(jnp.float32)
    pool_ref[...] = lax.dot_general(onehot, h_ref[...],
                                    (((0,), (0,)), ((), ())),
                                    precision=lax.Precision.HIGHEST,
                                    preferred_element_type=jnp.float32)


_pool = pl.pallas_call(
    _pool_body,
    out_shape=jax.ShapeDtypeStruct((NUM_GRAPHS, D), jnp.float32),
    in_specs=[pl.BlockSpec(memory_space=pltpu.VMEM)] * 2,
)


def kernel(x, edge_index, batch, W1, b1, W2, b2, g1, be1, g2, be2, eps_param):
    row = edge_index[0].astype(jnp.int32)
    col = edge_index[1].astype(jnp.int32)
    pad = EDGES_PAD - N_EDGES
    pad_rows = N_NODES + jnp.arange(pad, dtype=jnp.int32) % (N_ACC - N_NODES)
    pad_cols = jnp.arange(pad, dtype=jnp.int32) % N_NODES
    row_r = jnp.concatenate([row, pad_rows]).reshape(CPT, NW, CHUNK
                                                    ).transpose(1, 0, 2)
    col_r = jnp.concatenate([col, pad_cols]).reshape(CPT, NW, CHUNK
                                                     ).transpose(1, 0, 2)
    zeros = jnp.zeros((N_ACC, D), jnp.float32)
    batch2d = batch.astype(jnp.int32).reshape(N_NODES, 1)

    h = x
    pools = []
    for l in range(NUM_LAYERS):
        acc = _get_agg()(h, row_r, col_r, zeros)
        h = _mlp(eps_param[l].reshape(1, 1), acc, h,
                 W1[l], b1[l].reshape(1, D), W2[l], b2[l].reshape(1, D),
                 g1[l].reshape(1, D), be1[l].reshape(1, D),
                 g2[l].reshape(1, D), be2[l].reshape(1, D))
        pools.append(_pool(batch2d, h))
    return jnp.concatenate(pools, axis=-1)
